# baseline (speedup 1.0000x reference)
import jax
import jax.numpy as jnp
from jax.experimental import pallas as pl
from jax.experimental.pallas import tpu as pltpu

_N = 8192
_H = 32
_L = 3
_CUTOFF = 0.05
_BR = 256
_BC = 512


_BIG = 1e19


def _colsum_body(cb_ref, d_ref, out_ref):
    g = pl.program_id(0)

    @pl.when(g == 0)
    def _init():
        out_ref[...] = jnp.zeros_like(out_ref)

    c0 = cb_ref[0, 0]
    c1 = cb_ref[0, 1]
    c2 = cb_ref[0, 2]
    b0 = cb_ref[1, 0]
    b1 = cb_ref[1, 1]
    b2 = cb_ref[1, 2]

    for s in range(_N // _BC):
        def chunk(i, acc):
            a0, a1, a2 = acc
            d = d_ref[pl.ds(i * 8, 8), s * _BC : (s + 1) * _BC]
            q = jnp.where(d < _CUTOFF, 1.0 / d, _BIG)
            t0 = q - c0
            t1 = q - c1
            t2 = q - c2
            a0 = a0 + jax.lax.exp2((t0 * t0) * b0)
            a1 = a1 + jax.lax.exp2((t1 * t1) * b1)
            a2 = a2 + jax.lax.exp2((t2 * t2) * b2)
            return (a0, a1, a2)

        z = jnp.zeros((8, _BC), jnp.float32)
        a0, a1, a2 = jax.lax.fori_loop(0, _BR // 8, chunk, (z, z, z))
        cs = slice(s * _BC, (s + 1) * _BC)
        out_ref[0:1, cs] += jnp.sum(a0, axis=0, keepdims=True)
        out_ref[1:2, cs] += jnp.sum(a1, axis=0, keepdims=True)
        out_ref[2:3, cs] += jnp.sum(a2, axis=0, keepdims=True)

    dd = d_ref[:, pl.ds(g * _BR, _BR)]
    rl = jax.lax.broadcasted_iota(jnp.int32, (_BR, _BR), 0)
    cl = jax.lax.broadcasted_iota(jnp.int32, (_BR, _BR), 1)
    qd = jnp.where((rl == cl) & (dd < _CUTOFF), 1.0 / dd, _BIG)
    for l in range(_L):
        s = qd - cb_ref[0, l]
        w = jax.lax.exp2((s * s) * cb_ref[1, l])
        out_ref[l : l + 1, pl.ds(g * _BR, _BR)] -= jnp.sum(
            w, axis=0, keepdims=True
        )


def _head_body(z_ref, won_ref, bon_ref, wint_ref, bint_ref, cs_ref, wa_ref,
               bn_ref, ppp_ref, out_ppp_ref, out_reg_ref):
    z = z_ref[...]
    zcs = jnp.zeros((1, _H), jnp.float32)
    for l in range(_L):
        x = jnp.dot(z, won_ref[l], preferred_element_type=jnp.float32)
        x = x + bon_ref[l : l + 1, :]
        zos = jnp.maximum(x, 0.0) + jnp.log1p(jnp.exp(-jnp.abs(x)))
        s_on = jnp.sum(zos, axis=0, keepdims=True)
        cs = cs_ref[l : l + 1, :]
        v = jnp.dot(cs, zos, preferred_element_type=jnp.float32)
        inter = jnp.dot(v, wint_ref[l], preferred_element_type=jnp.float32)
        inter = inter + jnp.sum(cs) * bint_ref[l : l + 1, :]
        zcs = zcs + wa_ref[l : l + 1, :] * (s_on + inter) + bn_ref[0, l]
    out_ppp_ref[...] = ppp_ref[...] + 0.01 * zcs
    out_reg_ref[...] = 0.01 * jnp.sqrt(jnp.sum(zcs * zcs)).reshape(1, 1)


def kernel(geom_array, dist_matrix, ppp_params, W_on, B_on, W_int, B_int,
           mu, sigma, W_a, B_n):
    n = dist_matrix.shape[0]
    log2e = 1.4426950408889634
    cb = jnp.stack(
        [1.0 / mu, -log2e / (2.0 * sigma * sigma)]
    ).astype(jnp.float32)

    colsums = pl.pallas_call(
        _colsum_body,
        grid=(n // _BR,),
        in_specs=[
            pl.BlockSpec(memory_space=pltpu.SMEM),
            pl.BlockSpec((_BR, n), lambda g: (g, 0)),
        ],
        out_specs=pl.BlockSpec((_L, n), lambda g: (0, 0)),
        out_shape=jax.ShapeDtypeStruct((_L, n), jnp.float32),
    )(cb, dist_matrix)

    out_ppp, out_reg = pl.pallas_call(
        _head_body,
        in_specs=[
            pl.BlockSpec(memory_space=pltpu.VMEM),
            pl.BlockSpec(memory_space=pltpu.VMEM),
            pl.BlockSpec(memory_space=pltpu.VMEM),
            pl.BlockSpec(memory_space=pltpu.VMEM),
            pl.BlockSpec(memory_space=pltpu.VMEM),
            pl.BlockSpec(memory_space=pltpu.VMEM),
            pl.BlockSpec(memory_space=pltpu.VMEM),
            pl.BlockSpec(memory_space=pltpu.SMEM),
            pl.BlockSpec(memory_space=pltpu.VMEM),
        ],
        out_specs=[
            pl.BlockSpec(memory_space=pltpu.VMEM),
            pl.BlockSpec(memory_space=pltpu.VMEM),
        ],
        out_shape=[
            jax.ShapeDtypeStruct((1, _H), jnp.float32),
            jax.ShapeDtypeStruct((1, 1), jnp.float32),
        ],
    )(geom_array, W_on, B_on, W_int, B_int, colsums, W_a,
      B_n.reshape(1, _L), ppp_params.reshape(1, _H))

    return out_ppp.reshape(_H), out_reg.reshape(())


# speedup vs baseline: 1.8298x; 1.8298x over previous
import jax
import jax.numpy as jnp
from jax.experimental import pallas as pl
from jax.experimental.pallas import tpu as pltpu

_N = 8192
_H = 32
_L = 3
_CUTOFF = 0.05
_BR = 256
_BC = 512


_BIG = 1e19


def _colsum_body(cb_ref, d_ref, out_ref, q_ref):
    g = pl.program_id(0)

    @pl.when(g == 0)
    def _init():
        out_ref[...] = jnp.zeros_like(out_ref)

    d = d_ref[...]
    q_ref[...] = jnp.where(d < _CUTOFF, 1.0 / d, _BIG)
    q = q_ref[...]
    for l in range(_L):
        s = q - cb_ref[0, l]
        w = jax.lax.exp2((s * s) * cb_ref[1, l])
        out_ref[l : l + 1, :] += jnp.sum(w, axis=0, keepdims=True)

    dd = d_ref[:, pl.ds(g * _BR, _BR)]
    rl = jax.lax.broadcasted_iota(jnp.int32, (_BR, _BR), 0)
    cl = jax.lax.broadcasted_iota(jnp.int32, (_BR, _BR), 1)
    qd = jnp.where((rl == cl) & (dd < _CUTOFF), 1.0 / dd, _BIG)
    for l in range(_L):
        s = qd - cb_ref[0, l]
        w = jax.lax.exp2((s * s) * cb_ref[1, l])
        out_ref[l : l + 1, pl.ds(g * _BR, _BR)] -= jnp.sum(
            w, axis=0, keepdims=True
        )


def _head_body(z_ref, won_ref, bon_ref, wint_ref, bint_ref, cs_ref, wa_ref,
               bn_ref, ppp_ref, out_ppp_ref, out_reg_ref):
    z = z_ref[...]
    zcs = jnp.zeros((1, _H), jnp.float32)
    for l in range(_L):
        x = jnp.dot(z, won_ref[l], preferred_element_type=jnp.float32)
        x = x + bon_ref[l : l + 1, :]
        zos = jnp.maximum(x, 0.0) + jnp.log1p(jnp.exp(-jnp.abs(x)))
        s_on = jnp.sum(zos, axis=0, keepdims=True)
        cs = cs_ref[l : l + 1, :]
        v = jnp.dot(cs, zos, preferred_element_type=jnp.float32)
        inter = jnp.dot(v, wint_ref[l], preferred_element_type=jnp.float32)
        inter = inter + jnp.sum(cs) * bint_ref[l : l + 1, :]
        zcs = zcs + wa_ref[l : l + 1, :] * (s_on + inter) + bn_ref[0, l]
    out_ppp_ref[...] = ppp_ref[...] + 0.01 * zcs
    out_reg_ref[...] = 0.01 * jnp.sqrt(jnp.sum(zcs * zcs)).reshape(1, 1)


def kernel(geom_array, dist_matrix, ppp_params, W_on, B_on, W_int, B_int,
           mu, sigma, W_a, B_n):
    n = dist_matrix.shape[0]
    log2e = 1.4426950408889634
    cb = jnp.stack(
        [1.0 / mu, -log2e / (2.0 * sigma * sigma)]
    ).astype(jnp.float32)

    colsums = pl.pallas_call(
        _colsum_body,
        grid=(n // _BR,),
        in_specs=[
            pl.BlockSpec(memory_space=pltpu.SMEM),
            pl.BlockSpec((_BR, n), lambda g: (g, 0)),
        ],
        out_specs=pl.BlockSpec((_L, n), lambda g: (0, 0)),
        out_shape=jax.ShapeDtypeStruct((_L, n), jnp.float32),
        scratch_shapes=[pltpu.VMEM((_BR, n), jnp.float32)],
    )(cb, dist_matrix)

    out_ppp, out_reg = pl.pallas_call(
        _head_body,
        in_specs=[
            pl.BlockSpec(memory_space=pltpu.VMEM),
            pl.BlockSpec(memory_space=pltpu.VMEM),
            pl.BlockSpec(memory_space=pltpu.VMEM),
            pl.BlockSpec(memory_space=pltpu.VMEM),
            pl.BlockSpec(memory_space=pltpu.VMEM),
            pl.BlockSpec(memory_space=pltpu.VMEM),
            pl.BlockSpec(memory_space=pltpu.VMEM),
            pl.BlockSpec(memory_space=pltpu.SMEM),
            pl.BlockSpec(memory_space=pltpu.VMEM),
        ],
        out_specs=[
            pl.BlockSpec(memory_space=pltpu.VMEM),
            pl.BlockSpec(memory_space=pltpu.VMEM),
        ],
        out_shape=[
            jax.ShapeDtypeStruct((1, _H), jnp.float32),
            jax.ShapeDtypeStruct((1, 1), jnp.float32),
        ],
    )(geom_array, W_on, B_on, W_int, B_int, colsums, W_a,
      B_n.reshape(1, _L), ppp_params.reshape(1, _H))

    return out_ppp.reshape(_H), out_reg.reshape(())


# speedup vs baseline: 2.2919x; 1.2525x over previous
import jax
import jax.numpy as jnp
from jax.experimental import pallas as pl
from jax.experimental.pallas import tpu as pltpu

_N = 8192
_H = 32
_L = 3
_CUTOFF = 0.05
_BR = 256
_BC = 512


_BIG = 1e19


def _colsum_body(cb_ref, d_ref, out_ref, q_ref):
    g = pl.program_id(0)

    @pl.when(g == 0)
    def _init():
        out_ref[...] = jnp.zeros_like(out_ref)

    d = d_ref[...]
    q_ref[...] = jnp.where(d < _CUTOFF, 1.0 / d, _BIG)
    q = q_ref[...]
    ones = jnp.ones((1, _BR), jnp.float32)
    for l in range(_L):
        s = q - cb_ref[0, l]
        w = jax.lax.exp2((s * s) * cb_ref[1, l])
        out_ref[l : l + 1, :] += jnp.dot(
            ones, w, preferred_element_type=jnp.float32
        )

    dd = d_ref[:, pl.ds(g * _BR, _BR)]
    rl = jax.lax.broadcasted_iota(jnp.int32, (_BR, _BR), 0)
    cl = jax.lax.broadcasted_iota(jnp.int32, (_BR, _BR), 1)
    qd = jnp.where((rl == cl) & (dd < _CUTOFF), 1.0 / dd, _BIG)
    for l in range(_L):
        s = qd - cb_ref[0, l]
        w = jax.lax.exp2((s * s) * cb_ref[1, l])
        out_ref[l : l + 1, pl.ds(g * _BR, _BR)] -= jnp.sum(
            w, axis=0, keepdims=True
        )


def _head_body(z_ref, won_ref, bon_ref, wint_ref, bint_ref, cs_ref, wa_ref,
               bn_ref, ppp_ref, out_ppp_ref, out_reg_ref):
    z = z_ref[...]
    zcs = jnp.zeros((1, _H), jnp.float32)
    for l in range(_L):
        x = jnp.dot(z, won_ref[l], preferred_element_type=jnp.float32)
        x = x + bon_ref[l : l + 1, :]
        zos = jnp.maximum(x, 0.0) + jnp.log1p(jnp.exp(-jnp.abs(x)))
        s_on = jnp.sum(zos, axis=0, keepdims=True)
        cs = cs_ref[l : l + 1, :]
        v = jnp.dot(cs, zos, preferred_element_type=jnp.float32)
        inter = jnp.dot(v, wint_ref[l], preferred_element_type=jnp.float32)
        inter = inter + jnp.sum(cs) * bint_ref[l : l + 1, :]
        zcs = zcs + wa_ref[l : l + 1, :] * (s_on + inter) + bn_ref[0, l]
    out_ppp_ref[...] = ppp_ref[...] + 0.01 * zcs
    out_reg_ref[...] = 0.01 * jnp.sqrt(jnp.sum(zcs * zcs)).reshape(1, 1)


def kernel(geom_array, dist_matrix, ppp_params, W_on, B_on, W_int, B_int,
           mu, sigma, W_a, B_n):
    n = dist_matrix.shape[0]
    log2e = 1.4426950408889634
    cb = jnp.stack(
        [1.0 / mu, -log2e / (2.0 * sigma * sigma)]
    ).astype(jnp.float32)

    colsums = pl.pallas_call(
        _colsum_body,
        grid=(n // _BR,),
        in_specs=[
            pl.BlockSpec(memory_space=pltpu.SMEM),
            pl.BlockSpec((_BR, n), lambda g: (g, 0)),
        ],
        out_specs=pl.BlockSpec((_L, n), lambda g: (0, 0)),
        out_shape=jax.ShapeDtypeStruct((_L, n), jnp.float32),
        scratch_shapes=[pltpu.VMEM((_BR, n), jnp.float32)],
    )(cb, dist_matrix)

    out_ppp, out_reg = pl.pallas_call(
        _head_body,
        in_specs=[
            pl.BlockSpec(memory_space=pltpu.VMEM),
            pl.BlockSpec(memory_space=pltpu.VMEM),
            pl.BlockSpec(memory_space=pltpu.VMEM),
            pl.BlockSpec(memory_space=pltpu.VMEM),
            pl.BlockSpec(memory_space=pltpu.VMEM),
            pl.BlockSpec(memory_space=pltpu.VMEM),
            pl.BlockSpec(memory_space=pltpu.VMEM),
            pl.BlockSpec(memory_space=pltpu.SMEM),
            pl.BlockSpec(memory_space=pltpu.VMEM),
        ],
        out_specs=[
            pl.BlockSpec(memory_space=pltpu.VMEM),
            pl.BlockSpec(memory_space=pltpu.VMEM),
        ],
        out_shape=[
            jax.ShapeDtypeStruct((1, _H), jnp.float32),
            jax.ShapeDtypeStruct((1, 1), jnp.float32),
        ],
    )(geom_array, W_on, B_on, W_int, B_int, colsums, W_a,
      B_n.reshape(1, _L), ppp_params.reshape(1, _H))

    return out_ppp.reshape(_H), out_reg.reshape(())


# speedup vs baseline: 2.3554x; 1.0277x over previous
import jax
import jax.numpy as jnp
from jax.experimental import pallas as pl
from jax.experimental.pallas import tpu as pltpu

_N = 8192
_H = 32
_L = 3
_CUTOFF = 0.05
_BR = 256
_BC = 512


_BIG = 1e19


def _colsum_body(cb_ref, d_ref, out_ref, q_ref):
    g = pl.program_id(0)

    @pl.when(g == 0)
    def _init():
        out_ref[...] = jnp.zeros_like(out_ref)

    d = d_ref[...]
    q_ref[...] = jnp.where(d < _CUTOFF, 1.0 / d, _BIG)
    q = q_ref[...]
    ones = jnp.ones((1, _BR), jnp.bfloat16)
    for l in range(_L):
        s = q - cb_ref[0, l]
        w = jax.lax.exp2(((s * s) * cb_ref[1, l]).astype(jnp.bfloat16))
        out_ref[l : l + 1, :] += jnp.dot(
            ones, w, preferred_element_type=jnp.float32
        )

    dd = d_ref[:, pl.ds(g * _BR, _BR)]
    rl = jax.lax.broadcasted_iota(jnp.int32, (_BR, _BR), 0)
    cl = jax.lax.broadcasted_iota(jnp.int32, (_BR, _BR), 1)
    qd = jnp.where((rl == cl) & (dd < _CUTOFF), 1.0 / dd, _BIG)
    for l in range(_L):
        s = qd - cb_ref[0, l]
        w = jax.lax.exp2((s * s) * cb_ref[1, l])
        out_ref[l : l + 1, pl.ds(g * _BR, _BR)] -= jnp.sum(
            w, axis=0, keepdims=True
        )


def _head_body(z_ref, won_ref, bon_ref, wint_ref, bint_ref, cs_ref, wa_ref,
               bn_ref, ppp_ref, out_ppp_ref, out_reg_ref):
    z = z_ref[...]
    zcs = jnp.zeros((1, _H), jnp.float32)
    for l in range(_L):
        x = jnp.dot(z, won_ref[l], preferred_element_type=jnp.float32)
        x = x + bon_ref[l : l + 1, :]
        zos = jnp.maximum(x, 0.0) + jnp.log1p(jnp.exp(-jnp.abs(x)))
        s_on = jnp.sum(zos, axis=0, keepdims=True)
        cs = cs_ref[l : l + 1, :]
        v = jnp.dot(cs, zos, preferred_element_type=jnp.float32)
        inter = jnp.dot(v, wint_ref[l], preferred_element_type=jnp.float32)
        inter = inter + jnp.sum(cs) * bint_ref[l : l + 1, :]
        zcs = zcs + wa_ref[l : l + 1, :] * (s_on + inter) + bn_ref[0, l]
    out_ppp_ref[...] = ppp_ref[...] + 0.01 * zcs
    out_reg_ref[...] = 0.01 * jnp.sqrt(jnp.sum(zcs * zcs)).reshape(1, 1)


def kernel(geom_array, dist_matrix, ppp_params, W_on, B_on, W_int, B_int,
           mu, sigma, W_a, B_n):
    n = dist_matrix.shape[0]
    log2e = 1.4426950408889634
    cb = jnp.stack(
        [1.0 / mu, -log2e / (2.0 * sigma * sigma)]
    ).astype(jnp.float32)

    colsums = pl.pallas_call(
        _colsum_body,
        grid=(n // _BR,),
        in_specs=[
            pl.BlockSpec(memory_space=pltpu.SMEM),
            pl.BlockSpec((_BR, n), lambda g: (g, 0)),
        ],
        out_specs=pl.BlockSpec((_L, n), lambda g: (0, 0)),
        out_shape=jax.ShapeDtypeStruct((_L, n), jnp.float32),
        scratch_shapes=[pltpu.VMEM((_BR, n), jnp.float32)],
    )(cb, dist_matrix)

    out_ppp, out_reg = pl.pallas_call(
        _head_body,
        in_specs=[
            pl.BlockSpec(memory_space=pltpu.VMEM),
            pl.BlockSpec(memory_space=pltpu.VMEM),
            pl.BlockSpec(memory_space=pltpu.VMEM),
            pl.BlockSpec(memory_space=pltpu.VMEM),
            pl.BlockSpec(memory_space=pltpu.VMEM),
            pl.BlockSpec(memory_space=pltpu.VMEM),
            pl.BlockSpec(memory_space=pltpu.VMEM),
            pl.BlockSpec(memory_space=pltpu.SMEM),
            pl.BlockSpec(memory_space=pltpu.VMEM),
        ],
        out_specs=[
            pl.BlockSpec(memory_space=pltpu.VMEM),
            pl.BlockSpec(memory_space=pltpu.VMEM),
        ],
        out_shape=[
            jax.ShapeDtypeStruct((1, _H), jnp.float32),
            jax.ShapeDtypeStruct((1, 1), jnp.float32),
        ],
    )(geom_array, W_on, B_on, W_int, B_int, colsums, W_a,
      B_n.reshape(1, _L), ppp_params.reshape(1, _H))

    return out_ppp.reshape(_H), out_reg.reshape(())
